# baseline (speedup 1.0000x reference)
import functools

import jax
import jax.numpy as jnp
from jax import lax
from jax.experimental import pallas as pl
from jax.experimental.pallas import tpu as pltpu
from jax.experimental.pallas import tpu_sc as plsc

_B, _L, _D = 4096, 200, 64
_V = 1000
_N = _B * _L
_NW = 32
_ROWS_W = _N // _NW
_CHUNK = 256
_GRP = 16
_NCHUNK = _ROWS_W // _CHUNK
_STAGE = _CHUNK * _D


def _build():
    mesh = plsc.VectorSubcoreMesh(core_axis_name="c", subcore_axis_name="s")

    @functools.partial(
        pl.kernel,
        mesh=mesh,
        out_type=jax.ShapeDtypeStruct((_N * _D,), jnp.float32),
        compiler_params=pltpu.CompilerParams(
            use_tc_tiling_on_sc=False, needs_layout_passes=False),
        scratch_types=[
            pltpu.VMEM((_V * _D,), jnp.float32),
            pltpu.VMEM((_ROWS_W,), jnp.int32),
            pltpu.VMEM((_STAGE,), jnp.float32),
            pltpu.VMEM((_STAGE,), jnp.float32),
            pltpu.SemaphoreType.DMA,
            pltpu.SemaphoreType.DMA,
        ],
    )
    def gather_kernel(table_hbm, idx_hbm, out_hbm, table_v, idx_v,
                      stage_a, stage_b, sem_a, sem_b):
        wid = lax.axis_index("s") * 2 + lax.axis_index("c")
        rbase = wid * _ROWS_W
        pltpu.sync_copy(table_hbm, table_v)
        pltpu.sync_copy(idx_hbm.at[pl.ds(rbase, _ROWS_W)], idx_v)

        lane = lax.iota(jnp.int32, 16)
        lane64 = lane * _D

        def fill(chunk_i, stage):
            @plsc.parallel_loop(0, _CHUNK // _GRP)
            def grp(g):
                rows = idx_v[pl.ds(chunk_i * _CHUNK + g * _GRP, _GRP)]
                toff = rows * _D
                soff = lane64 + g * (_GRP * _D)
                for d in range(_D):
                    dvec = jnp.bitwise_and(lane + d, _D - 1)
                    v = plsc.load_gather(table_v, [toff + dvec])
                    plsc.store_scatter(stage, [soff + dvec], v)

        def out_slice(chunk_i):
            return out_hbm.at[pl.ds((rbase + chunk_i * _CHUNK) * _D, _STAGE)]

        fill(0, stage_a)
        pltpu.async_copy(stage_a, out_slice(0), sem_a)
        fill(1, stage_b)
        pltpu.async_copy(stage_b, out_slice(1), sem_b)

        def body(i, carry):
            pltpu.make_async_copy(stage_a, out_slice(2 * i), sem_a).wait()
            fill(2 * i, stage_a)
            pltpu.async_copy(stage_a, out_slice(2 * i), sem_a)
            pltpu.make_async_copy(stage_b, out_slice(2 * i + 1), sem_b).wait()
            fill(2 * i + 1, stage_b)
            pltpu.async_copy(stage_b, out_slice(2 * i + 1), sem_b)
            return carry

        lax.fori_loop(1, _NCHUNK // 2, body, 0)
        pltpu.make_async_copy(stage_a, out_slice(0), sem_a).wait()
        pltpu.make_async_copy(stage_b, out_slice(1), sem_b).wait()

    return gather_kernel


_gather = _build()


def kernel(idx, node_type_embed, degree_embed):
    idx0 = idx[:, :, 0].reshape(_N)
    out = _gather(node_type_embed.reshape(_V * _D), idx0)
    return out.reshape(_B, _L, _D)


# speedup vs baseline: 1.2579x; 1.2579x over previous
import functools

import jax
import jax.numpy as jnp
from jax import lax
from jax.experimental import pallas as pl
from jax.experimental.pallas import tpu as pltpu
from jax.experimental.pallas import tpu_sc as plsc

_B, _L, _D = 4096, 200, 64
_V = 1000
_N = _B * _L
_NW = 32
_ROWS_W = _N // _NW
_NIT = 100
_SCH = 144
_VCH = 112
_GRP = 16
_SROWS_W = _SCH * _NIT
_VROWS_W = _VCH * _NIT


def _build():
    mesh = plsc.VectorSubcoreMesh(core_axis_name="c", subcore_axis_name="s")

    @functools.partial(
        pl.kernel,
        mesh=mesh,
        out_type=jax.ShapeDtypeStruct((_N, _D), jnp.float32),
        compiler_params=pltpu.CompilerParams(
            use_tc_tiling_on_sc=False, needs_layout_passes=False),
        scratch_types=[
            pltpu.VMEM((_V, _D), jnp.float32),
            pltpu.VMEM((_ROWS_W,), jnp.int32),
            pltpu.VMEM((_SCH, _D), jnp.float32),
            pltpu.VMEM((_SCH, _D), jnp.float32),
            pltpu.VMEM((_VCH, _D), jnp.float32),
            pltpu.VMEM((_VCH, _D), jnp.float32),
            pltpu.VMEM_SHARED((_V, _D), jnp.float32),
            pltpu.SemaphoreType.DMA,
            pltpu.SemaphoreType.DMA,
            pltpu.SemaphoreType.DMA,
            pltpu.SemaphoreType.DMA,
            pltpu.SemaphoreType.DMA,
            pltpu.SemaphoreType.DMA,
        ],
    )
    def gather_kernel(table_hbm, idx_hbm, out_hbm, table_v, idx_v,
                      sbuf_a, sbuf_b, vbuf_a, vbuf_b, table_sp,
                      gsem_a, gsem_b, ssem_a, ssem_b, vsem_a, vsem_b):
        sid = lax.axis_index("s")
        wid = sid * 2 + lax.axis_index("c")
        rbase = wid * _ROWS_W
        sbase = rbase
        vbase = rbase + _SROWS_W

        @pl.when(sid == 0)
        def _():
            pltpu.sync_copy(table_hbm, table_sp)

        pltpu.sync_copy(table_hbm, table_v)
        pltpu.sync_copy(idx_hbm.at[pl.ds(rbase, _ROWS_W)], idx_v)
        plsc.subcore_barrier()

        lane = lax.iota(jnp.int32, 16)

        def fill(c, stage):
            @plsc.parallel_loop(0, _VCH // _GRP)
            def grp(g):
                rows = idx_v[pl.ds(_SROWS_W + c * _VCH + g * _GRP, _GRP)]
                srow = lane + g * _GRP
                for d in range(_D):
                    dvec = jnp.bitwise_and(lane + d, _D - 1)
                    v = plsc.load_gather(table_v, [rows, dvec])
                    plsc.store_scatter(stage, [srow, dvec], v)

        def step(c, sbuf, vbuf, gsem, ssem, vsem, steady):
            if steady:
                pltpu.make_async_copy(
                    sbuf, out_hbm.at[pl.ds(sbase, _SCH)], ssem).wait()
                pltpu.make_async_copy(
                    vbuf, out_hbm.at[pl.ds(vbase, _VCH)], vsem).wait()
            cp = pltpu.async_copy(
                table_sp.at[idx_v.at[pl.ds(c * _SCH, _SCH)]], sbuf, gsem)
            fill(c, vbuf)
            cp.wait()
            pltpu.async_copy(sbuf, out_hbm.at[pl.ds(sbase + c * _SCH, _SCH)],
                             ssem)
            pltpu.async_copy(vbuf, out_hbm.at[pl.ds(vbase + c * _VCH, _VCH)],
                             vsem)

        step(0, sbuf_a, vbuf_a, gsem_a, ssem_a, vsem_a, False)
        step(1, sbuf_b, vbuf_b, gsem_b, ssem_b, vsem_b, False)

        def body(i, carry):
            step(2 * i, sbuf_a, vbuf_a, gsem_a, ssem_a, vsem_a, True)
            step(2 * i + 1, sbuf_b, vbuf_b, gsem_b, ssem_b, vsem_b, True)
            return carry

        lax.fori_loop(1, _NIT // 2, body, 0)
        pltpu.make_async_copy(sbuf_a, out_hbm.at[pl.ds(sbase, _SCH)],
                              ssem_a).wait()
        pltpu.make_async_copy(vbuf_a, out_hbm.at[pl.ds(vbase, _VCH)],
                              vsem_a).wait()
        pltpu.make_async_copy(sbuf_b, out_hbm.at[pl.ds(sbase, _SCH)],
                              ssem_b).wait()
        pltpu.make_async_copy(vbuf_b, out_hbm.at[pl.ds(vbase, _VCH)],
                              vsem_b).wait()

    return gather_kernel


_gather = _build()


def kernel(idx, node_type_embed, degree_embed):
    idx0 = idx[:, :, 0].reshape(_N)
    out = _gather(node_type_embed, idx0)
    return out.reshape(_B, _L, _D)
